# baseline (speedup 1.0000x reference)
import jax
import jax.numpy as jnp
import numpy as np
from jax import lax
from jax.experimental import pallas as pl
from jax.experimental.pallas import tpu as pltpu
from jax.experimental.pallas import tpu_sc as plsc

B = 131072
INPUT_DIM = 3
NUM_LEVELS = 16
LEVEL_DIM = 2
BASE_RES = 16
T = 1 << 19
IDX_MASK = 63
P1 = np.array(2654435761, np.uint32).view(np.int32).item()
P2 = 805459861

NW = 32
PPW = B // NW
C = 1024
NCHUNK = PPW // C
NG = C // 16
N_DIRECT = 3
OUT_DIM = NUM_LEVELS * LEVEL_DIM


L0_SIZE = (BASE_RES + 1) ** 3
L1_SIZE = (2 * BASE_RES + 1) ** 3
N_LOCAL = 2


def _grid_body(inp_hbm, tab_hbm, out_hbm, inp_v, idx_a, idx_b, w_a, w_b,
               fp_a, fp_b, l0_v, l1_v, out_v, sem_a, sem_b):
    wid = lax.axis_index("s") * 2 + lax.axis_index("c")
    lane = lax.iota(jnp.int32, 16)
    idx_bufs = (idx_a, idx_b)
    w_bufs = (w_a, w_b)
    fp_bufs = (fp_a, fp_b)
    sems = (sem_a, sem_b)
    local_tabs = (l0_v, l1_v)

    pltpu.sync_copy(tab_hbm.at[pl.ds(0, L0_SIZE)], l0_v)
    pltpu.sync_copy(tab_hbm.at[pl.ds(T, L1_SIZE)], l1_v)

    def idx_level(l, idx_ref, w_ref):
        direct = l < N_DIRECT
        res = BASE_RES << l
        scale = jnp.float32(res)
        base_l = l * T
        if direct:
            s1 = res + 1
            s2 = s1 * s1

        def idx_body(g, carry):
            s = g * 16
            x = inp_v[pl.ds(s, 16)]
            y = inp_v[pl.ds(C + s, 16)]
            z = inp_v[pl.ds(2 * C + s, 16)]
            px = x * scale
            py = y * scale
            pz = z * scale
            ix = px.astype(jnp.int32)
            iy = py.astype(jnp.int32)
            iz = pz.astype(jnp.int32)
            fx = px - ix.astype(jnp.float32)
            fy = py - iy.astype(jnp.float32)
            fz = pz - iz.astype(jnp.float32)
            gx = 1.0 - fx
            gy = 1.0 - fy
            gz = 1.0 - fz
            wxy = (gx * gy, fx * gy, gx * fy, fx * fy)
            if direct:
                t0 = ix + s1 * iy + s2 * iz + base_l
            else:
                hy0 = iy * P1
                hz0 = iz * P2
            for c in range(8):
                if direct:
                    d = (c & 1) + (s1 if c & 2 else 0) + (s2 if c & 4 else 0)
                    idx = t0 + d
                else:
                    hx = ix + 1 if c & 1 else ix
                    hy = hy0 + P1 if c & 2 else hy0
                    hz = hz0 + P2 if c & 4 else hz0
                    idx = ((hx ^ hy ^ hz) & IDX_MASK) + base_l
                w = wxy[c & 3] * (fz if c & 4 else gz)
                idx_ref[pl.ds(c * C + s, 16)] = idx
                w_ref[pl.ds(c * C + s, 16)] = w
            return carry

        lax.fori_loop(0, NG, idx_body, 0)

    def acc_level(l, w_ref, fp_ref):
        def acc_body(g, carry):
            s = g * 16
            acc0 = jnp.zeros((16,), jnp.float32)
            acc1 = jnp.zeros((16,), jnp.float32)
            for c in range(8):
                w = w_ref[pl.ds(c * C + s, 16)]
                r = fp_ref[pl.ds(c * C + s, 16)]
                f0 = lax.bitcast_convert_type(r << 16, jnp.float32)
                f1 = lax.bitcast_convert_type(r & jnp.int32(-65536),
                                              jnp.float32)
                acc0 = acc0 + w * f0
                acc1 = acc1 + w * f1
            oidx = (s + lane) * OUT_DIM + 2 * l
            plsc.store_scatter(out_v, [oidx], acc0)
            plsc.store_scatter(out_v, [oidx + 1], acc1)
            return carry

        lax.fori_loop(0, NG, acc_body, 0)

    def fused_local_level(l):
        tab_v = local_tabs[l]
        res = BASE_RES << l
        scale = jnp.float32(res)
        s1 = res + 1
        s2 = s1 * s1

        def body(g, carry):
            s = g * 16
            x = inp_v[pl.ds(s, 16)]
            y = inp_v[pl.ds(C + s, 16)]
            z = inp_v[pl.ds(2 * C + s, 16)]
            px = x * scale
            py = y * scale
            pz = z * scale
            ix = px.astype(jnp.int32)
            iy = py.astype(jnp.int32)
            iz = pz.astype(jnp.int32)
            fx = px - ix.astype(jnp.float32)
            fy = py - iy.astype(jnp.float32)
            fz = pz - iz.astype(jnp.float32)
            gx = 1.0 - fx
            gy = 1.0 - fy
            gz = 1.0 - fz
            wxy = (gx * gy, fx * gy, gx * fy, fx * fy)
            t0 = ix + s1 * iy + s2 * iz
            acc0 = jnp.zeros((16,), jnp.float32)
            acc1 = jnp.zeros((16,), jnp.float32)
            for c in range(8):
                d = (c & 1) + (s1 if c & 2 else 0) + (s2 if c & 4 else 0)
                r = plsc.load_gather(tab_v, [t0 + d])
                w = wxy[c & 3] * (fz if c & 4 else gz)
                f0 = lax.bitcast_convert_type(r << 16, jnp.float32)
                f1 = lax.bitcast_convert_type(r & jnp.int32(-65536),
                                              jnp.float32)
                acc0 = acc0 + w * f0
                acc1 = acc1 + w * f1
            oidx = (s + lane) * OUT_DIM + 2 * l
            plsc.store_scatter(out_v, [oidx], acc0)
            plsc.store_scatter(out_v, [oidx + 1], acc1)
            return carry

        lax.fori_loop(0, NG, body, 0)

    def fire(l):
        p = l % 2
        return pltpu.async_copy(tab_hbm.at[idx_bufs[p]], fp_bufs[p], sems[p])

    def chunk_body(k, carry):
        base = wid * PPW + k * C
        for d in range(INPUT_DIM):
            pltpu.sync_copy(inp_hbm.at[pl.ds(d * B + base, C)],
                            inp_v.at[pl.ds(d * C, C)])
        idx_level(N_LOCAL, idx_bufs[N_LOCAL % 2], w_bufs[N_LOCAL % 2])
        handles = {N_LOCAL: fire(N_LOCAL)}
        fused_local_level(0)
        fused_local_level(1)
        for l in range(N_LOCAL, NUM_LEVELS):
            if l + 1 < NUM_LEVELS:
                idx_level(l + 1, idx_bufs[(l + 1) % 2], w_bufs[(l + 1) % 2])
                handles[l + 1] = fire(l + 1)
            handles[l].wait()
            acc_level(l, w_bufs[l % 2], fp_bufs[l % 2])
        pltpu.sync_copy(out_v, out_hbm.at[pl.ds(base * OUT_DIM, C * OUT_DIM)])
        return carry

    lax.fori_loop(0, NCHUNK, chunk_body, 0)


@jax.jit
def kernel(inputs, table):
    tab_bf16 = table.astype(jnp.bfloat16).reshape(NUM_LEVELS * T, LEVEL_DIM)
    tab = lax.bitcast_convert_type(tab_bf16, jnp.int32)
    inp_flat = inputs.T.reshape(INPUT_DIM * B)
    mesh = plsc.VectorSubcoreMesh(core_axis_name="c", subcore_axis_name="s")
    f = pl.kernel(
        _grid_body,
        out_type=jax.ShapeDtypeStruct((B * OUT_DIM,), jnp.float32),
        mesh=mesh,
        compiler_params=pltpu.CompilerParams(needs_layout_passes=False,
                                             use_tc_tiling_on_sc=False),
        scratch_types=[
            pltpu.VMEM((C * INPUT_DIM,), jnp.float32),
            pltpu.VMEM((8 * C,), jnp.int32),
            pltpu.VMEM((8 * C,), jnp.int32),
            pltpu.VMEM((8 * C,), jnp.float32),
            pltpu.VMEM((8 * C,), jnp.float32),
            pltpu.VMEM((8 * C,), jnp.int32),
            pltpu.VMEM((8 * C,), jnp.int32),
            pltpu.VMEM((L0_SIZE,), jnp.int32),
            pltpu.VMEM((L1_SIZE,), jnp.int32),
            pltpu.VMEM((C * OUT_DIM,), jnp.float32),
            pltpu.SemaphoreType.DMA,
            pltpu.SemaphoreType.DMA,
        ],
    )
    return f(inp_flat, tab).reshape(B, OUT_DIM)


# speedup vs baseline: 34.5001x; 34.5001x over previous
import jax
import jax.numpy as jnp
import numpy as np
from jax import lax
from jax.experimental import pallas as pl
from jax.experimental.pallas import tpu as pltpu
from jax.experimental.pallas import tpu_sc as plsc

B = 131072
INPUT_DIM = 3
NUM_LEVELS = 16
LEVEL_DIM = 2
BASE_RES = 16
T = 1 << 19
IDX_MASK = T - 1
P1 = np.array(2654435761, np.uint32).view(np.int32).item()
P2 = 805459861

NW = 32
PPW = B // NW
C = 1024
NCHUNK = PPW // C
NG = C // 16
N_DIRECT = 3
OUT_DIM = NUM_LEVELS * LEVEL_DIM


L0_SIZE = (BASE_RES + 1) ** 3
L1_SIZE = (2 * BASE_RES + 1) ** 3
N_LOCAL = 2


def _grid_body(inp_hbm, tab_hbm, out_hbm, inp_v, idx_a, idx_b, w_a, w_b,
               fp_a, fp_b, l0_v, l1_v, out_v, sem_a, sem_b):
    wid = lax.axis_index("s") * 2 + lax.axis_index("c")
    lane = lax.iota(jnp.int32, 16)
    idx_bufs = (idx_a, idx_b)
    w_bufs = (w_a, w_b)
    fp_bufs = (fp_a, fp_b)
    sems = (sem_a, sem_b)
    local_tabs = (l0_v, l1_v)

    pltpu.sync_copy(tab_hbm.at[pl.ds(0, L0_SIZE)], l0_v)
    pltpu.sync_copy(tab_hbm.at[pl.ds(T, L1_SIZE)], l1_v)

    def idx_level(l, idx_ref, w_ref):
        direct = l < N_DIRECT
        res = BASE_RES << l
        scale = jnp.float32(res)
        base_l = l * T
        if direct:
            s1 = res + 1
            s2 = s1 * s1

        def idx_body(g, carry):
            s = g * 16
            x = inp_v[pl.ds(s, 16)]
            y = inp_v[pl.ds(C + s, 16)]
            z = inp_v[pl.ds(2 * C + s, 16)]
            px = x * scale
            py = y * scale
            pz = z * scale
            ix = px.astype(jnp.int32)
            iy = py.astype(jnp.int32)
            iz = pz.astype(jnp.int32)
            fx = px - ix.astype(jnp.float32)
            fy = py - iy.astype(jnp.float32)
            fz = pz - iz.astype(jnp.float32)
            gx = 1.0 - fx
            gy = 1.0 - fy
            gz = 1.0 - fz
            wxy = (gx * gy, fx * gy, gx * fy, fx * fy)
            if direct:
                t0 = ix + s1 * iy + s2 * iz + base_l
            else:
                hy0 = iy * P1
                hz0 = iz * P2
            for c in range(8):
                if direct:
                    d = (c & 1) + (s1 if c & 2 else 0) + (s2 if c & 4 else 0)
                    idx = t0 + d
                else:
                    hx = ix + 1 if c & 1 else ix
                    hy = hy0 + P1 if c & 2 else hy0
                    hz = hz0 + P2 if c & 4 else hz0
                    idx = ((hx ^ hy ^ hz) & IDX_MASK) + base_l
                w = wxy[c & 3] * (fz if c & 4 else gz)
                idx_ref[pl.ds(c * C + s, 16)] = idx
                w_ref[pl.ds(c * C + s, 16)] = w
            return carry

        lax.fori_loop(0, NG, idx_body, 0)

    def acc_level(l, w_ref, fp_ref):
        def acc_body(g, carry):
            s = g * 16
            acc0 = jnp.zeros((16,), jnp.float32)
            acc1 = jnp.zeros((16,), jnp.float32)
            for c in range(8):
                w = w_ref[pl.ds(c * C + s, 16)]
                r = fp_ref[pl.ds(c * C + s, 16)]
                f0 = lax.bitcast_convert_type(r << 16, jnp.float32)
                f1 = lax.bitcast_convert_type(r & jnp.int32(-65536),
                                              jnp.float32)
                acc0 = acc0 + w * f0
                acc1 = acc1 + w * f1
            oidx = (s + lane) * OUT_DIM + 2 * l
            plsc.store_scatter(out_v, [oidx], acc0)
            plsc.store_scatter(out_v, [oidx + 1], acc1)
            return carry

        lax.fori_loop(0, NG, acc_body, 0)

    def fused_local_level(l):
        tab_v = local_tabs[l]
        res = BASE_RES << l
        scale = jnp.float32(res)
        s1 = res + 1
        s2 = s1 * s1

        def body(g, carry):
            s = g * 16
            x = inp_v[pl.ds(s, 16)]
            y = inp_v[pl.ds(C + s, 16)]
            z = inp_v[pl.ds(2 * C + s, 16)]
            px = x * scale
            py = y * scale
            pz = z * scale
            ix = px.astype(jnp.int32)
            iy = py.astype(jnp.int32)
            iz = pz.astype(jnp.int32)
            fx = px - ix.astype(jnp.float32)
            fy = py - iy.astype(jnp.float32)
            fz = pz - iz.astype(jnp.float32)
            gx = 1.0 - fx
            gy = 1.0 - fy
            gz = 1.0 - fz
            wxy = (gx * gy, fx * gy, gx * fy, fx * fy)
            t0 = ix + s1 * iy + s2 * iz
            acc0 = jnp.zeros((16,), jnp.float32)
            acc1 = jnp.zeros((16,), jnp.float32)
            for c in range(8):
                d = (c & 1) + (s1 if c & 2 else 0) + (s2 if c & 4 else 0)
                r = plsc.load_gather(tab_v, [t0 + d])
                w = wxy[c & 3] * (fz if c & 4 else gz)
                f0 = lax.bitcast_convert_type(r << 16, jnp.float32)
                f1 = lax.bitcast_convert_type(r & jnp.int32(-65536),
                                              jnp.float32)
                acc0 = acc0 + w * f0
                acc1 = acc1 + w * f1
            oidx = (s + lane) * OUT_DIM + 2 * l
            plsc.store_scatter(out_v, [oidx], acc0)
            plsc.store_scatter(out_v, [oidx + 1], acc1)
            return carry

        lax.fori_loop(0, NG, body, 0)

    def fire(l):
        p = l % 2
        return pltpu.async_copy(tab_hbm.at[idx_bufs[p]], fp_bufs[p], sems[p])

    def chunk_body(k, carry):
        base = wid * PPW + k * C
        for d in range(INPUT_DIM):
            pltpu.sync_copy(inp_hbm.at[pl.ds(d * B + base, C)],
                            inp_v.at[pl.ds(d * C, C)])
        idx_level(N_LOCAL, idx_bufs[N_LOCAL % 2], w_bufs[N_LOCAL % 2])
        handles = {N_LOCAL: fire(N_LOCAL)}
        fused_local_level(0)
        fused_local_level(1)
        for l in range(N_LOCAL, NUM_LEVELS):
            if l + 1 < NUM_LEVELS:
                idx_level(l + 1, idx_bufs[(l + 1) % 2], w_bufs[(l + 1) % 2])
                handles[l + 1] = fire(l + 1)
            handles[l].wait()
            acc_level(l, w_bufs[l % 2], fp_bufs[l % 2])
        pltpu.sync_copy(out_v, out_hbm.at[pl.ds(base * OUT_DIM, C * OUT_DIM)])
        return carry

    lax.fori_loop(0, NCHUNK, chunk_body, 0)


@jax.jit
def kernel(inputs, table):
    tab_bf16 = table.astype(jnp.bfloat16).reshape(NUM_LEVELS * T, LEVEL_DIM)
    tab = lax.bitcast_convert_type(tab_bf16, jnp.int32)
    inp_flat = inputs.T.reshape(INPUT_DIM * B)
    mesh = plsc.VectorSubcoreMesh(core_axis_name="c", subcore_axis_name="s")
    f = pl.kernel(
        _grid_body,
        out_type=jax.ShapeDtypeStruct((B * OUT_DIM,), jnp.float32),
        mesh=mesh,
        compiler_params=pltpu.CompilerParams(needs_layout_passes=False,
                                             use_tc_tiling_on_sc=False),
        scratch_types=[
            pltpu.VMEM((C * INPUT_DIM,), jnp.float32),
            pltpu.VMEM((8 * C,), jnp.int32),
            pltpu.VMEM((8 * C,), jnp.int32),
            pltpu.VMEM((8 * C,), jnp.float32),
            pltpu.VMEM((8 * C,), jnp.float32),
            pltpu.VMEM((8 * C,), jnp.int32),
            pltpu.VMEM((8 * C,), jnp.int32),
            pltpu.VMEM((L0_SIZE,), jnp.int32),
            pltpu.VMEM((L1_SIZE,), jnp.int32),
            pltpu.VMEM((C * OUT_DIM,), jnp.float32),
            pltpu.SemaphoreType.DMA,
            pltpu.SemaphoreType.DMA,
        ],
    )
    return f(inp_flat, tab).reshape(B, OUT_DIM)
